# baseline (speedup 1.0000x reference)
import functools

import jax
import jax.numpy as jnp
from jax import lax
from jax.experimental import pallas as pl
from jax.experimental.pallas import tpu as pltpu
from jax.experimental.pallas import tpu_sc as plsc

KNB = 16
B_ = 5
N_ = 2000
NT = B_ * N_
NPAD = 10240
NCOL = 2048
NEDGE = float(NT * KNB)

NW = 32
NPW = NPAD // NW
CH = 8
NCHUNK = NPW // CH
FG = 128

RT = 256
NTL = 128
NST = NPAD // NTL


GRP = 8


def _knn_body(pts_ref, ptsT_ref, out_ref, dm_ref):
    b = pl.program_id(0)
    rt = pl.program_id(1)
    p = pts_ref[0]
    pT = ptsT_ref[0]
    dm_ref[...] = jnp.dot(p, pT, preferred_element_type=jnp.float32)
    sq_c = pT[0:1] * pT[0:1] + pT[1:2] * pT[1:2] + pT[2:3] * pT[2:3]
    col16 = lax.broadcasted_iota(jnp.int32, (GRP, KNB), 1)
    colid = lax.broadcasted_iota(jnp.int32, (GRP, NCOL), 1)

    def grp_body(g, carry):
        pg = pts_ref[0, pl.ds(g * GRP, GRP), :]
        sq_r = (pg[:, 0:1] * pg[:, 0:1] + pg[:, 1:2] * pg[:, 1:2]
                + pg[:, 2:3] * pg[:, 2:3])
        dotg = dm_ref[pl.ds(g * GRP, GRP), :]
        d = (sq_r + sq_c) - 2.0 * dotg
        rowid = (rt * RT + g * GRP
                 + lax.broadcasted_iota(jnp.int32, (GRP, NCOL), 0))
        d = d + jnp.where(colid == rowid, jnp.float32(1e10), jnp.float32(0.0))
        d = jnp.where(colid >= N_, jnp.float32(jnp.inf), d)
        idxacc = jnp.zeros((GRP, KNB), jnp.int32)
        for t in range(KNB):
            m = jnp.min(d, axis=1, keepdims=True)
            am = jnp.min(jnp.where(d == m, colid, jnp.int32(2**30)),
                         axis=1, keepdims=True)
            idxacc = jnp.where(col16 == t, am + b * N_, idxacc)
            d = jnp.where(colid == am, jnp.float32(jnp.inf), d)
        out_ref[0, pl.ds(g * GRP, GRP), :] = idxacc
        return carry

    lax.fori_loop(0, RT // GRP, grp_body, 0)


def _knn(pos_pad, posT, *, interpret=False):
    return pl.pallas_call(
        _knn_body,
        grid=(B_, NCOL // RT),
        in_specs=[
            pl.BlockSpec((1, RT, 8), lambda b, r: (b, r, 0)),
            pl.BlockSpec((1, 8, NCOL), lambda b, r: (b, 0, 0)),
        ],
        out_specs=pl.BlockSpec((1, RT, KNB), lambda b, r: (b, r, 0)),
        out_shape=jax.ShapeDtypeStruct((B_, NCOL, KNB), jnp.int32),
        scratch_shapes=[pltpu.VMEM((RT, NCOL), jnp.float32)],
        interpret=interpret,
    )(pos_pad, posT)


def _normu_body(norm, fin, fpad, fout, y_ref, st_ref, g_ref, be_ref,
                wt_ref, bv_ref, x_ref, u_ref):
    y = y_ref[...]
    if norm:
        st = st_ref[...]
        mean = jnp.sum(st[:, 0], axis=0, keepdims=True) / NEDGE
        msq = jnp.sum(st[:, 1], axis=0, keepdims=True) / NEDGE
        var = msq - mean * mean
        scale = g_ref[...] / jnp.sqrt(var + 1e-5)
        xn = (y - mean) * scale + be_ref[...]
    else:
        xn = y
    x_ref[:, 0:fin] = xn
    if fpad > fin:
        x_ref[:, fin:fpad] = jnp.zeros((x_ref.shape[0], fpad - fin),
                                       jnp.float32)
    u_ref[...] = jnp.dot(xn, wt_ref[...],
                         preferred_element_type=jnp.float32) + bv_ref[...]


def _normu(y, st, g, be, wt, bv, fin, fpad, fout, norm, *, interpret=False):
    MT = 1024
    body = functools.partial(_normu_body, norm, fin, fpad, fout)
    return pl.pallas_call(
        body,
        grid=(NPAD // MT,),
        in_specs=[
            pl.BlockSpec((MT, fin), lambda i: (i, 0)),
            pl.BlockSpec((NST, 2, fin), lambda i: (0, 0, 0)),
            pl.BlockSpec((1, fin), lambda i: (0, 0)),
            pl.BlockSpec((1, fin), lambda i: (0, 0)),
            pl.BlockSpec((fin, fout), lambda i: (0, 0)),
            pl.BlockSpec((1, fout), lambda i: (0, 0)),
        ],
        out_specs=[
            pl.BlockSpec((MT, fpad), lambda i: (i, 0)),
            pl.BlockSpec((MT, fout), lambda i: (i, 0)),
        ],
        out_shape=[
            jax.ShapeDtypeStruct((NPAD, fpad), jnp.float32),
            jax.ShapeDtypeStruct((NPAD, fout), jnp.float32),
        ],
        interpret=interpret,
    )(y, st, g, be, wt, bv)


NFW = KNB * NPAD // NW
GCH = 128
NGCH = NFW // GCH
NBUF = 4


@functools.lru_cache(maxsize=None)
def _make_gather_kernel():
    mesh = plsc.VectorSubcoreMesh(core_axis_name="c", subcore_axis_name="s")

    @functools.partial(
        pl.kernel,
        mesh=mesh,
        out_type=jax.ShapeDtypeStruct((KNB, NPAD, FG), jnp.float32),
        scratch_types=[
            pltpu.VMEM((NFW,), jnp.int32),
            pltpu.VMEM((NBUF, GCH, FG), jnp.float32),
            pltpu.SemaphoreType.DMA, pltpu.SemaphoreType.DMA,
            pltpu.SemaphoreType.DMA, pltpu.SemaphoreType.DMA,
            pltpu.SemaphoreType.DMA, pltpu.SemaphoreType.DMA,
            pltpu.SemaphoreType.DMA, pltpu.SemaphoreType.DMA,
        ],
    )
    def gather(idx_hbm, x_hbm, xj_hbm, idx_v, z_v,
               g0, g1, g2, g3, w0, w1, w2, w3):
        wid = lax.axis_index("s") * 2 + lax.axis_index("c")
        fbase = wid * NFW
        kslab = wid // 2
        row0 = (wid % 2) * NFW
        gsems = (g0, g1, g2, g3)
        wsems = (w0, w1, w2, w3)
        pltpu.sync_copy(idx_hbm.at[pl.ds(fbase, NFW)], idx_v)

        def fire_gather(ch, b):
            pltpu.async_copy(
                x_hbm.at[idx_v.at[pl.ds(ch * GCH, GCH)]],
                z_v.at[b], gsems[b])

        def wait_gather(b):
            pltpu.make_async_copy(
                x_hbm.at[pl.ds(0, GCH)], z_v.at[b], gsems[b]).wait()

        def fire_write(ch, b):
            pltpu.async_copy(
                z_v.at[b],
                xj_hbm.at[kslab, pl.ds(row0 + ch * GCH, GCH), :],
                wsems[b])

        def wait_write(b):
            pltpu.make_async_copy(
                z_v.at[b], xj_hbm.at[0, pl.ds(0, GCH), :], wsems[b]).wait()

        for b in range(NBUF - 1):
            fire_gather(b, b)

        def round_body(r, carry):
            for s in range(NBUF):
                ch = r * NBUF + s
                nx = ch + NBUF - 1
                bnx = (s + NBUF - 1) % NBUF

                @pl.when(nx < NGCH)
                def _():
                    @pl.when(nx >= NBUF)
                    def _():
                        wait_write(bnx)
                    fire_gather(nx, bnx)

                wait_gather(s)
                fire_write(ch, s)
            return carry

        lax.fori_loop(0, NGCH // NBUF, round_body, 0)
        for b in range(NBUF):
            wait_write(b)

    return gather


def _reduce_body(fout, u_ref, d_ref, x_ref, wb_ref, y_ref, st_ref):
    nt = pl.program_id(0)
    u = u_ref[...]
    x = x_ref[...]
    wb = wb_ref[...]
    nodeid = nt * NTL + lax.broadcasted_iota(jnp.int32, (NTL, fout), 0)
    valid = nodeid < NT
    macc = None
    s = None
    ss = None
    for k in range(KNB):
        v = jnp.dot(d_ref[k] - x, wb, preferred_element_type=jnp.float32)
        m = u + v
        lr = jnp.where(m >= 0, m, 0.2 * m)
        lrm = jnp.where(valid, lr, 0.0)
        sk = jnp.sum(lrm, axis=0, keepdims=True)
        ssk = jnp.sum(lrm * lrm, axis=0, keepdims=True)
        if k == 0:
            macc, s, ss = lr, sk, ssk
        else:
            macc = jnp.maximum(macc, lr)
            s = s + sk
            ss = ss + ssk
    y_ref[...] = macc
    st_ref[0, 0:1] = s
    st_ref[0, 1:2] = ss


def _reduce(u, xj, x, wb, fout, *, interpret=False):
    body = functools.partial(_reduce_body, fout)
    return pl.pallas_call(
        body,
        grid=(NST,),
        in_specs=[
            pl.BlockSpec((NTL, fout), lambda nt: (nt, 0)),
            pl.BlockSpec((KNB, NTL, FG), lambda nt: (0, nt, 0)),
            pl.BlockSpec((NTL, FG), lambda nt: (nt, 0)),
            pl.BlockSpec((FG, fout), lambda nt: (0, 0)),
        ],
        out_specs=[
            pl.BlockSpec((NTL, fout), lambda nt: (nt, 0)),
            pl.BlockSpec((1, 2, fout), lambda nt: (nt, 0, 0)),
        ],
        out_shape=[
            jax.ShapeDtypeStruct((NPAD, fout), jnp.float32),
            jax.ShapeDtypeStruct((NST, 2, fout), jnp.float32),
        ],
        interpret=interpret,
    )(u, xj, x, wb)


def _final_body(y1_ref, y2_ref, y3_ref, st1_ref, st2_ref, st3_ref,
                g1_ref, be1_ref, g2_ref, be2_ref, g3_ref, be3_ref,
                wf1_ref, bf1_ref, gf_ref, bef_ref, wf2_ref, bf2_ref,
                out_ref, gm1, gm2, gm3):
    b = pl.program_id(0)

    def norm_of(st_ref, g_ref, be_ref, v):
        st = st_ref[...]
        mean = jnp.sum(st[:, 0], axis=0, keepdims=True) / NEDGE
        msq = jnp.sum(st[:, 1], axis=0, keepdims=True) / NEDGE
        var = msq - mean * mean
        scale = g_ref[...] / jnp.sqrt(var + 1e-5)
        return (v - mean) * scale + be_ref[...]

    for y_ref, st_ref, g_ref, be_ref, gm in (
            (y1_ref, st1_ref, g1_ref, be1_ref, gm1),
            (y2_ref, st2_ref, g2_ref, be2_ref, gm2),
            (y3_ref, st3_ref, g3_ref, be3_ref, gm3)):
        mx = jnp.max(y_ref[...], axis=0, keepdims=True)
        gn = norm_of(st_ref, g_ref, be_ref, mx)
        rows = lax.broadcasted_iota(jnp.int32, gm.shape, 0)
        gm[...] = jnp.where(rows == b, jnp.broadcast_to(gn, gm.shape),
                            gm[...])

    @pl.when(b == B_ - 1)
    def _():
        wf1 = wf1_ref[...]
        h = (jnp.dot(gm1[...], wf1[0:64], preferred_element_type=jnp.float32)
             + jnp.dot(gm2[...], wf1[64:192],
                       preferred_element_type=jnp.float32)
             + jnp.dot(gm3[...], wf1[192:448],
                       preferred_element_type=jnp.float32)) + bf1_ref[...]
        h = jnp.where(h >= 0, h, 0.2 * h)
        rows = lax.broadcasted_iota(jnp.int32, h.shape, 0)
        valid = rows < B_
        hm = jnp.where(valid, h, 0.0)
        mean = jnp.sum(hm, axis=0, keepdims=True) / float(B_)
        diff = h - mean
        var = jnp.sum(jnp.where(valid, diff * diff, 0.0), axis=0,
                      keepdims=True) / float(B_)
        hn = gf_ref[...] * diff / jnp.sqrt(var + 1e-5) + bef_ref[...]
        out_ref[...] = jnp.dot(hn, wf2_ref[...],
                               preferred_element_type=jnp.float32) + bf2_ref[...]


def _final(y1, y2, y3, st1, st2, st3, g1, be1, g2, be2, g3, be3,
           wf1, bf1, gf, bef, wf2, bf2, *, interpret=False):
    full = lambda shape: pl.BlockSpec(shape, lambda b: tuple(0 for _ in shape))
    return pl.pallas_call(
        _final_body,
        grid=(B_,),
        in_specs=[
            pl.BlockSpec((N_, 64), lambda b: (b, 0)),
            pl.BlockSpec((N_, 128), lambda b: (b, 0)),
            pl.BlockSpec((N_, 256), lambda b: (b, 0)),
            full((NST, 2, 64)), full((NST, 2, 128)), full((NST, 2, 256)),
            full((1, 64)), full((1, 64)),
            full((1, 128)), full((1, 128)),
            full((1, 256)), full((1, 256)),
            full((448, 512)), full((1, 512)), full((1, 512)), full((1, 512)),
            full((512, 128)), full((1, 128)),
        ],
        out_specs=pl.BlockSpec((8, 128), lambda b: (0, 0)),
        out_shape=jax.ShapeDtypeStruct((8, 128), jnp.float32),
        scratch_shapes=[
            pltpu.VMEM((8, 64), jnp.float32),
            pltpu.VMEM((8, 128), jnp.float32),
            pltpu.VMEM((8, 256), jnp.float32),
        ],
        interpret=interpret,
    )(y1, y2, y3, st1, st2, st3, g1, be1, g2, be2, g3, be3,
      wf1, bf1, gf, bef, wf2, bf2)


def _row2(v):
    return v.reshape(1, -1)


def kernel(points, W1, b1, g1, be1, W2, b2, g2, be2, W3, b3, g3, be3,
           Wf1, bf1, gf, bef, Wf2, bf2):
    pos = points[..., :3]
    pos_pad = jnp.zeros((B_, NCOL, 8), jnp.float32).at[:, :N_, :3].set(pos)
    posT = jnp.transpose(pos_pad, (0, 2, 1))

    idx = _knn(pos_pad, posT)
    idx = idx[:, :N_].reshape(NT, KNB)
    idx_pad = jnp.pad(idx, ((0, NPAD - NT), (0, 0)))
    idx_km = jnp.transpose(idx_pad, (1, 0)).reshape(-1)

    x0p = jnp.pad(points.reshape(NT, 5), ((0, NPAD - NT), (0, FG - 5)))
    dummy_st = jnp.zeros((NST, 2, FG), jnp.float32)
    one = jnp.ones((1, FG), jnp.float32)
    zero = jnp.zeros((1, FG), jnp.float32)

    Wt1p = jnp.pad(W1[:5], ((0, FG - 5), (0, 0)))
    Wb1p = jnp.pad(W1[5:], ((0, FG - 5), (0, 0)))
    Wt2, Wb2 = W2[:64], jnp.pad(W2[64:], ((0, 64), (0, 0)))
    Wt3, Wb3 = W3[:128], W3[128:]

    gather = _make_gather_kernel()

    _, U1 = _normu(x0p, dummy_st, one, zero, Wt1p, _row2(b1),
                   FG, FG, 64, False)
    J1 = gather(idx_km, x0p)
    Y1, ST1 = _reduce(U1, J1, x0p, Wb1p, 64)

    X2, U2 = _normu(Y1, ST1, _row2(g1), _row2(be1), Wt2, _row2(b2),
                    64, FG, 128, True)
    J2 = gather(idx_km, X2)
    Y2, ST2 = _reduce(U2, J2, X2, Wb2, 128)

    X3, U3 = _normu(Y2, ST2, _row2(g2), _row2(be2), Wt3, _row2(b3),
                    128, FG, 256, True)
    J3 = gather(idx_km, X3)
    Y3, ST3 = _reduce(U3, J3, X3, Wb3, 256)

    out8 = _final(Y1[:NT], Y2[:NT], Y3[:NT], ST1, ST2, ST3,
                  _row2(g1), _row2(be1), _row2(g2), _row2(be2),
                  _row2(g3), _row2(be3),
                  Wf1, _row2(bf1), _row2(gf), _row2(bef), Wf2, _row2(bf2))
    return out8[:B_]


# speedup vs baseline: 3.7230x; 3.7230x over previous
import functools

import jax
import jax.numpy as jnp
from jax import lax
from jax.experimental import pallas as pl
from jax.experimental.pallas import tpu as pltpu
from jax.experimental.pallas import tpu_sc as plsc

KNB = 16
B_ = 5
N_ = 2000
NT = B_ * N_
NPAD = 10240
NCOL = 2048
NEDGE = float(NT * KNB)

NW = 32
NPW = NPAD // NW
CH = 8
NCHUNK = NPW // CH
FG = 128

RT = 64
NTL = 128
NST = NPAD // NTL


GRP = 8


def _knn_body(pts_ref, ptsT_ref, out_ref):
    b = pl.program_id(0)
    rt = pl.program_id(1)
    p = pts_ref[0]
    pT = ptsT_ref[0]
    dotm = jnp.dot(p, pT, preferred_element_type=jnp.float32)
    sq_c = pT[0:1] * pT[0:1] + pT[1:2] * pT[1:2] + pT[2:3] * pT[2:3]
    col16 = lax.broadcasted_iota(jnp.int32, (GRP, KNB), 1)
    colid = lax.broadcasted_iota(jnp.int32, (GRP, NCOL), 1)
    for g in range(RT // GRP):
        pg = p[g * GRP:(g + 1) * GRP]
        sq_r = (pg[:, 0:1] * pg[:, 0:1] + pg[:, 1:2] * pg[:, 1:2]
                + pg[:, 2:3] * pg[:, 2:3])
        d = (sq_r + sq_c) - 2.0 * dotm[g * GRP:(g + 1) * GRP]
        rowid = (rt * RT + g * GRP
                 + lax.broadcasted_iota(jnp.int32, (GRP, NCOL), 0))
        d = d + jnp.where(colid == rowid, jnp.float32(1e10), jnp.float32(0.0))
        d = jnp.where(colid >= N_, jnp.float32(jnp.inf), d)
        idxacc = jnp.zeros((GRP, KNB), jnp.int32)
        for t in range(KNB):
            m = jnp.min(d, axis=1, keepdims=True)
            am = jnp.min(jnp.where(d == m, colid, jnp.int32(2**30)),
                         axis=1, keepdims=True)
            idxacc = jnp.where(col16 == t, am + b * N_, idxacc)
            d = jnp.where(colid == am, jnp.float32(jnp.inf), d)
        out_ref[0, g * GRP:(g + 1) * GRP, :] = idxacc


def _knn(pos_pad, posT, *, interpret=False):
    return pl.pallas_call(
        _knn_body,
        grid=(B_, NCOL // RT),
        in_specs=[
            pl.BlockSpec((1, RT, 8), lambda b, r: (b, r, 0)),
            pl.BlockSpec((1, 8, NCOL), lambda b, r: (b, 0, 0)),
        ],
        out_specs=pl.BlockSpec((1, RT, KNB), lambda b, r: (b, r, 0)),
        out_shape=jax.ShapeDtypeStruct((B_, NCOL, KNB), jnp.int32),
        interpret=interpret,
    )(pos_pad, posT)


def _normu_body(norm, fin, fpad, fout, y_ref, st_ref, g_ref, be_ref,
                wt_ref, bv_ref, x_ref, u_ref):
    y = y_ref[...]
    if norm:
        st = st_ref[...]
        mean = jnp.sum(st[:, 0], axis=0, keepdims=True) / NEDGE
        msq = jnp.sum(st[:, 1], axis=0, keepdims=True) / NEDGE
        var = msq - mean * mean
        scale = g_ref[...] / jnp.sqrt(var + 1e-5)
        xn = (y - mean) * scale + be_ref[...]
    else:
        xn = y
    x_ref[:, 0:fin] = xn
    if fpad > fin:
        x_ref[:, fin:fpad] = jnp.zeros((x_ref.shape[0], fpad - fin),
                                       jnp.float32)
    u_ref[...] = jnp.dot(xn, wt_ref[...],
                         preferred_element_type=jnp.float32) + bv_ref[...]


def _normu(y, st, g, be, wt, bv, fin, fpad, fout, norm, *, interpret=False):
    MT = 1024
    body = functools.partial(_normu_body, norm, fin, fpad, fout)
    return pl.pallas_call(
        body,
        grid=(NPAD // MT,),
        in_specs=[
            pl.BlockSpec((MT, fin), lambda i: (i, 0)),
            pl.BlockSpec((NST, 2, fin), lambda i: (0, 0, 0)),
            pl.BlockSpec((1, fin), lambda i: (0, 0)),
            pl.BlockSpec((1, fin), lambda i: (0, 0)),
            pl.BlockSpec((fin, fout), lambda i: (0, 0)),
            pl.BlockSpec((1, fout), lambda i: (0, 0)),
        ],
        out_specs=[
            pl.BlockSpec((MT, fpad), lambda i: (i, 0)),
            pl.BlockSpec((MT, fout), lambda i: (i, 0)),
        ],
        out_shape=[
            jax.ShapeDtypeStruct((NPAD, fpad), jnp.float32),
            jax.ShapeDtypeStruct((NPAD, fout), jnp.float32),
        ],
        interpret=interpret,
    )(y, st, g, be, wt, bv)


NFW = KNB * NPAD // NW
GCH = 128
NGCH = NFW // GCH
NBUF = 4


@functools.lru_cache(maxsize=None)
def _make_gather_kernel():
    mesh = plsc.VectorSubcoreMesh(core_axis_name="c", subcore_axis_name="s")

    @functools.partial(
        pl.kernel,
        mesh=mesh,
        out_type=jax.ShapeDtypeStruct((KNB, NPAD, FG), jnp.float32),
        scratch_types=[
            pltpu.VMEM((NFW,), jnp.int32),
            pltpu.VMEM((NBUF, GCH, FG), jnp.float32),
            pltpu.SemaphoreType.DMA, pltpu.SemaphoreType.DMA,
            pltpu.SemaphoreType.DMA, pltpu.SemaphoreType.DMA,
            pltpu.SemaphoreType.DMA, pltpu.SemaphoreType.DMA,
            pltpu.SemaphoreType.DMA, pltpu.SemaphoreType.DMA,
        ],
    )
    def gather(idx_hbm, x_hbm, xj_hbm, idx_v, z_v,
               g0, g1, g2, g3, w0, w1, w2, w3):
        wid = lax.axis_index("s") * 2 + lax.axis_index("c")
        fbase = wid * NFW
        kslab = wid // 2
        row0 = (wid % 2) * NFW
        gsems = (g0, g1, g2, g3)
        wsems = (w0, w1, w2, w3)
        pltpu.sync_copy(idx_hbm.at[pl.ds(fbase, NFW)], idx_v)

        def fire_gather(ch, b):
            pltpu.async_copy(
                x_hbm.at[idx_v.at[pl.ds(ch * GCH, GCH)]],
                z_v.at[b], gsems[b])

        def wait_gather(b):
            pltpu.make_async_copy(
                x_hbm.at[pl.ds(0, GCH)], z_v.at[b], gsems[b]).wait()

        def fire_write(ch, b):
            pltpu.async_copy(
                z_v.at[b],
                xj_hbm.at[kslab, pl.ds(row0 + ch * GCH, GCH), :],
                wsems[b])

        def wait_write(b):
            pltpu.make_async_copy(
                z_v.at[b], xj_hbm.at[0, pl.ds(0, GCH), :], wsems[b]).wait()

        for b in range(NBUF - 1):
            fire_gather(b, b)

        def round_body(r, carry):
            for s in range(NBUF):
                ch = r * NBUF + s
                nx = ch + NBUF - 1
                bnx = (s + NBUF - 1) % NBUF

                @pl.when(nx < NGCH)
                def _():
                    @pl.when(nx >= NBUF)
                    def _():
                        wait_write(bnx)
                    fire_gather(nx, bnx)

                wait_gather(s)
                fire_write(ch, s)
            return carry

        lax.fori_loop(0, NGCH // NBUF, round_body, 0)
        for b in range(NBUF):
            wait_write(b)

    return gather


def _reduce_body(fout, u_ref, d_ref, x_ref, wb_ref, y_ref, st_ref):
    nt = pl.program_id(0)
    u = u_ref[...]
    x = x_ref[...]
    wb = wb_ref[...]
    nodeid = nt * NTL + lax.broadcasted_iota(jnp.int32, (NTL, fout), 0)
    valid = nodeid < NT
    macc = None
    s = None
    ss = None
    for k in range(KNB):
        v = jnp.dot(d_ref[k] - x, wb, preferred_element_type=jnp.float32)
        m = u + v
        lr = jnp.where(m >= 0, m, 0.2 * m)
        lrm = jnp.where(valid, lr, 0.0)
        sk = jnp.sum(lrm, axis=0, keepdims=True)
        ssk = jnp.sum(lrm * lrm, axis=0, keepdims=True)
        if k == 0:
            macc, s, ss = lr, sk, ssk
        else:
            macc = jnp.maximum(macc, lr)
            s = s + sk
            ss = ss + ssk
    y_ref[...] = macc
    st_ref[0, 0:1] = s
    st_ref[0, 1:2] = ss


def _reduce(u, xj, x, wb, fout, *, interpret=False):
    body = functools.partial(_reduce_body, fout)
    return pl.pallas_call(
        body,
        grid=(NST,),
        in_specs=[
            pl.BlockSpec((NTL, fout), lambda nt: (nt, 0)),
            pl.BlockSpec((KNB, NTL, FG), lambda nt: (0, nt, 0)),
            pl.BlockSpec((NTL, FG), lambda nt: (nt, 0)),
            pl.BlockSpec((FG, fout), lambda nt: (0, 0)),
        ],
        out_specs=[
            pl.BlockSpec((NTL, fout), lambda nt: (nt, 0)),
            pl.BlockSpec((1, 2, fout), lambda nt: (nt, 0, 0)),
        ],
        out_shape=[
            jax.ShapeDtypeStruct((NPAD, fout), jnp.float32),
            jax.ShapeDtypeStruct((NST, 2, fout), jnp.float32),
        ],
        interpret=interpret,
    )(u, xj, x, wb)


def _final_body(y1_ref, y2_ref, y3_ref, st1_ref, st2_ref, st3_ref,
                g1_ref, be1_ref, g2_ref, be2_ref, g3_ref, be3_ref,
                wf1_ref, bf1_ref, gf_ref, bef_ref, wf2_ref, bf2_ref,
                out_ref, gm1, gm2, gm3):
    b = pl.program_id(0)

    def norm_of(st_ref, g_ref, be_ref, v):
        st = st_ref[...]
        mean = jnp.sum(st[:, 0], axis=0, keepdims=True) / NEDGE
        msq = jnp.sum(st[:, 1], axis=0, keepdims=True) / NEDGE
        var = msq - mean * mean
        scale = g_ref[...] / jnp.sqrt(var + 1e-5)
        return (v - mean) * scale + be_ref[...]

    for y_ref, st_ref, g_ref, be_ref, gm in (
            (y1_ref, st1_ref, g1_ref, be1_ref, gm1),
            (y2_ref, st2_ref, g2_ref, be2_ref, gm2),
            (y3_ref, st3_ref, g3_ref, be3_ref, gm3)):
        mx = jnp.max(y_ref[...], axis=0, keepdims=True)
        gn = norm_of(st_ref, g_ref, be_ref, mx)
        rows = lax.broadcasted_iota(jnp.int32, gm.shape, 0)
        gm[...] = jnp.where(rows == b, jnp.broadcast_to(gn, gm.shape),
                            gm[...])

    @pl.when(b == B_ - 1)
    def _():
        wf1 = wf1_ref[...]
        h = (jnp.dot(gm1[...], wf1[0:64], preferred_element_type=jnp.float32)
             + jnp.dot(gm2[...], wf1[64:192],
                       preferred_element_type=jnp.float32)
             + jnp.dot(gm3[...], wf1[192:448],
                       preferred_element_type=jnp.float32)) + bf1_ref[...]
        h = jnp.where(h >= 0, h, 0.2 * h)
        rows = lax.broadcasted_iota(jnp.int32, h.shape, 0)
        valid = rows < B_
        hm = jnp.where(valid, h, 0.0)
        mean = jnp.sum(hm, axis=0, keepdims=True) / float(B_)
        diff = h - mean
        var = jnp.sum(jnp.where(valid, diff * diff, 0.0), axis=0,
                      keepdims=True) / float(B_)
        hn = gf_ref[...] * diff / jnp.sqrt(var + 1e-5) + bef_ref[...]
        out_ref[...] = jnp.dot(hn, wf2_ref[...],
                               preferred_element_type=jnp.float32) + bf2_ref[...]


def _final(y1, y2, y3, st1, st2, st3, g1, be1, g2, be2, g3, be3,
           wf1, bf1, gf, bef, wf2, bf2, *, interpret=False):
    full = lambda shape: pl.BlockSpec(shape, lambda b: tuple(0 for _ in shape))
    return pl.pallas_call(
        _final_body,
        grid=(B_,),
        in_specs=[
            pl.BlockSpec((N_, 64), lambda b: (b, 0)),
            pl.BlockSpec((N_, 128), lambda b: (b, 0)),
            pl.BlockSpec((N_, 256), lambda b: (b, 0)),
            full((NST, 2, 64)), full((NST, 2, 128)), full((NST, 2, 256)),
            full((1, 64)), full((1, 64)),
            full((1, 128)), full((1, 128)),
            full((1, 256)), full((1, 256)),
            full((448, 512)), full((1, 512)), full((1, 512)), full((1, 512)),
            full((512, 128)), full((1, 128)),
        ],
        out_specs=pl.BlockSpec((8, 128), lambda b: (0, 0)),
        out_shape=jax.ShapeDtypeStruct((8, 128), jnp.float32),
        scratch_shapes=[
            pltpu.VMEM((8, 64), jnp.float32),
            pltpu.VMEM((8, 128), jnp.float32),
            pltpu.VMEM((8, 256), jnp.float32),
        ],
        interpret=interpret,
    )(y1, y2, y3, st1, st2, st3, g1, be1, g2, be2, g3, be3,
      wf1, bf1, gf, bef, wf2, bf2)


def _row2(v):
    return v.reshape(1, -1)


def kernel(points, W1, b1, g1, be1, W2, b2, g2, be2, W3, b3, g3, be3,
           Wf1, bf1, gf, bef, Wf2, bf2):
    pos = points[..., :3]
    pos_pad = jnp.zeros((B_, NCOL, 8), jnp.float32).at[:, :N_, :3].set(pos)
    posT = jnp.transpose(pos_pad, (0, 2, 1))

    idx = _knn(pos_pad, posT)
    idx = idx[:, :N_].reshape(NT, KNB)
    idx_pad = jnp.pad(idx, ((0, NPAD - NT), (0, 0)))
    idx_km = jnp.transpose(idx_pad, (1, 0)).reshape(-1)

    x0p = jnp.pad(points.reshape(NT, 5), ((0, NPAD - NT), (0, FG - 5)))
    dummy_st = jnp.zeros((NST, 2, FG), jnp.float32)
    one = jnp.ones((1, FG), jnp.float32)
    zero = jnp.zeros((1, FG), jnp.float32)

    Wt1p = jnp.pad(W1[:5], ((0, FG - 5), (0, 0)))
    Wb1p = jnp.pad(W1[5:], ((0, FG - 5), (0, 0)))
    Wt2, Wb2 = W2[:64], jnp.pad(W2[64:], ((0, 64), (0, 0)))
    Wt3, Wb3 = W3[:128], W3[128:]

    gather = _make_gather_kernel()

    _, U1 = _normu(x0p, dummy_st, one, zero, Wt1p, _row2(b1),
                   FG, FG, 64, False)
    J1 = gather(idx_km, x0p)
    Y1, ST1 = _reduce(U1, J1, x0p, Wb1p, 64)

    X2, U2 = _normu(Y1, ST1, _row2(g1), _row2(be1), Wt2, _row2(b2),
                    64, FG, 128, True)
    J2 = gather(idx_km, X2)
    Y2, ST2 = _reduce(U2, J2, X2, Wb2, 128)

    X3, U3 = _normu(Y2, ST2, _row2(g2), _row2(be2), Wt3, _row2(b3),
                    128, FG, 256, True)
    J3 = gather(idx_km, X3)
    Y3, ST3 = _reduce(U3, J3, X3, Wb3, 256)

    out8 = _final(Y1[:NT], Y2[:NT], Y3[:NT], ST1, ST2, ST3,
                  _row2(g1), _row2(be1), _row2(g2), _row2(be2),
                  _row2(g3), _row2(be3),
                  Wf1, _row2(bf1), _row2(gf), _row2(bef), Wf2, _row2(bf2))
    return out8[:B_]


# speedup vs baseline: 4.7597x; 1.2784x over previous
import functools

import jax
import jax.numpy as jnp
from jax import lax
from jax.experimental import pallas as pl
from jax.experimental.pallas import tpu as pltpu
from jax.experimental.pallas import tpu_sc as plsc

KNB = 16
B_ = 5
N_ = 2000
NT = B_ * N_
NPAD = 10240
NCOL = 2048
NEDGE = float(NT * KNB)

NW = 32
NPW = NPAD // NW
CH = 8
NCHUNK = NPW // CH
FG = 128

RT = 256
NTL = 256
NST = NPAD // NTL


GRP = 8


def _knn_body(pts_ref, ptsT_ref, sqr_ref, sqc_ref, out_ref):
    b = pl.program_id(0)
    rt = pl.program_id(1)
    p = pts_ref[0]
    pT = ptsT_ref[0]
    dotm = jnp.dot(p, pT, preferred_element_type=jnp.float32)
    sq_r = sqr_ref[0]
    sq_c = sqc_ref[0]
    d = (sq_r + sq_c) - 2.0 * dotm
    rowid = rt * RT + lax.broadcasted_iota(jnp.int32, (RT, NCOL), 0)
    colid = lax.broadcasted_iota(jnp.int32, (RT, NCOL), 1)
    d = d + jnp.where(colid == rowid, jnp.float32(1e10), jnp.float32(0.0))
    d = jnp.where(colid >= N_, jnp.float32(jnp.inf), d)
    idxacc = jnp.zeros((RT, KNB), jnp.int32)
    col16 = lax.broadcasted_iota(jnp.int32, (RT, KNB), 1)
    for t in range(KNB):
        m = jnp.min(d, axis=1, keepdims=True)
        am = jnp.min(jnp.where(d == m, colid, jnp.int32(2**30)),
                     axis=1, keepdims=True)
        idxacc = jnp.where(col16 == t, am + b * N_, idxacc)
        d = jnp.where(colid == am, jnp.float32(jnp.inf), d)
    out_ref[0] = idxacc


def _knn(pos_pad, posT, sqr, sqc, *, interpret=False):
    return pl.pallas_call(
        _knn_body,
        grid=(B_, NCOL // RT),
        in_specs=[
            pl.BlockSpec((1, RT, 8), lambda b, r: (b, r, 0)),
            pl.BlockSpec((1, 8, NCOL), lambda b, r: (b, 0, 0)),
            pl.BlockSpec((1, RT, 1), lambda b, r: (b, r, 0)),
            pl.BlockSpec((1, 1, NCOL), lambda b, r: (b, 0, 0)),
        ],
        out_specs=pl.BlockSpec((1, RT, KNB), lambda b, r: (b, r, 0)),
        out_shape=jax.ShapeDtypeStruct((B_, NCOL, KNB), jnp.int32),
        interpret=interpret,
    )(pos_pad, posT, sqr, sqc)


def _normu_body(norm, fin, fpad, fout, y_ref, st_ref, g_ref, be_ref,
                wt_ref, bv_ref, x_ref, u_ref):
    y = y_ref[...]
    if norm:
        st = st_ref[...]
        mean = jnp.sum(st[:, 0], axis=0, keepdims=True) / NEDGE
        msq = jnp.sum(st[:, 1], axis=0, keepdims=True) / NEDGE
        var = msq - mean * mean
        xn = g_ref[...] * (y - mean) / jnp.sqrt(var + 1e-5) + be_ref[...]
    else:
        xn = y
    x_ref[:, 0:fin] = xn
    if fpad > fin:
        x_ref[:, fin:fpad] = jnp.zeros((x_ref.shape[0], fpad - fin),
                                       jnp.float32)
    u_ref[...] = jnp.dot(xn, wt_ref[...],
                         preferred_element_type=jnp.float32) + bv_ref[...]


def _normu(y, st, g, be, wt, bv, fin, fpad, fout, norm, *, interpret=False):
    MT = 1024
    body = functools.partial(_normu_body, norm, fin, fpad, fout)
    return pl.pallas_call(
        body,
        grid=(NPAD // MT,),
        in_specs=[
            pl.BlockSpec((MT, fin), lambda i: (i, 0)),
            pl.BlockSpec((NST, 2, fin), lambda i: (0, 0, 0)),
            pl.BlockSpec((1, fin), lambda i: (0, 0)),
            pl.BlockSpec((1, fin), lambda i: (0, 0)),
            pl.BlockSpec((fin, fout), lambda i: (0, 0)),
            pl.BlockSpec((1, fout), lambda i: (0, 0)),
        ],
        out_specs=[
            pl.BlockSpec((MT, fpad), lambda i: (i, 0)),
            pl.BlockSpec((MT, fout), lambda i: (i, 0)),
        ],
        out_shape=[
            jax.ShapeDtypeStruct((NPAD, fpad), jnp.float32),
            jax.ShapeDtypeStruct((NPAD, fout), jnp.float32),
        ],
        interpret=interpret,
    )(y, st, g, be, wt, bv)


NFW = KNB * NPAD // NW
GCH = 128
NGCH = NFW // GCH
NBUF = 5


@functools.lru_cache(maxsize=None)
def _make_gather_kernel():
    mesh = plsc.VectorSubcoreMesh(core_axis_name="c", subcore_axis_name="s")

    @functools.partial(
        pl.kernel,
        mesh=mesh,
        out_type=jax.ShapeDtypeStruct((KNB, NPAD, FG), jnp.float32),
        scratch_types=[
            pltpu.VMEM((NFW,), jnp.int32),
            pltpu.VMEM((NBUF, GCH, FG), jnp.float32),
            pltpu.SemaphoreType.DMA, pltpu.SemaphoreType.DMA,
            pltpu.SemaphoreType.DMA, pltpu.SemaphoreType.DMA,
            pltpu.SemaphoreType.DMA, pltpu.SemaphoreType.DMA,
            pltpu.SemaphoreType.DMA, pltpu.SemaphoreType.DMA,
            pltpu.SemaphoreType.DMA, pltpu.SemaphoreType.DMA,
        ],
    )
    def gather(idx_hbm, x_hbm, xj_hbm, idx_v, z_v,
               g0, g1, g2, g3, g4, w0, w1, w2, w3, w4):
        wid = lax.axis_index("s") * 2 + lax.axis_index("c")
        fbase = wid * NFW
        kslab = wid // 2
        row0 = (wid % 2) * NFW
        gsems = (g0, g1, g2, g3, g4)
        wsems = (w0, w1, w2, w3, w4)
        pltpu.sync_copy(idx_hbm.at[pl.ds(fbase, NFW)], idx_v)

        def fire_gather(ch, b):
            pltpu.async_copy(
                x_hbm.at[idx_v.at[pl.ds(ch * GCH, GCH)]],
                z_v.at[b], gsems[b])

        def wait_gather(b):
            pltpu.make_async_copy(
                x_hbm.at[pl.ds(0, GCH)], z_v.at[b], gsems[b]).wait()

        def fire_write(ch, b):
            pltpu.async_copy(
                z_v.at[b],
                xj_hbm.at[kslab, pl.ds(row0 + ch * GCH, GCH), :],
                wsems[b])

        def wait_write(b):
            pltpu.make_async_copy(
                z_v.at[b], xj_hbm.at[0, pl.ds(0, GCH), :], wsems[b]).wait()

        for b in range(NBUF - 1):
            fire_gather(b, b)

        def round_body(r, carry):
            for s in range(NBUF):
                ch = r * NBUF + s
                nx = ch + NBUF - 1
                bnx = (s + NBUF - 1) % NBUF

                @pl.when(nx < NGCH)
                def _():
                    @pl.when(nx >= NBUF)
                    def _():
                        wait_write(bnx)
                    fire_gather(nx, bnx)

                wait_gather(s)
                fire_write(ch, s)
            return carry

        lax.fori_loop(0, NGCH // NBUF, round_body, 0)
        for b in range(NBUF):
            wait_write(b)

    return gather


def _reduce_body(fout, u_ref, d_ref, x_ref, wb_ref, y_ref, st_ref):
    nt = pl.program_id(0)
    u = u_ref[...]
    x = x_ref[...]
    wb = wb_ref[...]
    nodeid = nt * NTL + lax.broadcasted_iota(jnp.int32, (NTL, fout), 0)
    valid = nodeid < NT
    macc = None
    s = None
    ss = None
    for k in range(KNB):
        v = jnp.dot(d_ref[k] - x, wb, preferred_element_type=jnp.float32)
        m = u + v
        lr = jnp.where(m >= 0, m, 0.2 * m)
        lrm = jnp.where(valid, lr, 0.0)
        sk = jnp.sum(lrm, axis=0, keepdims=True)
        ssk = jnp.sum(lrm * lrm, axis=0, keepdims=True)
        if k == 0:
            macc, s, ss = lr, sk, ssk
        else:
            macc = jnp.maximum(macc, lr)
            s = s + sk
            ss = ss + ssk
    y_ref[...] = macc
    st_ref[0, 0:1] = s
    st_ref[0, 1:2] = ss


def _reduce(u, xj, x, wb, fout, *, interpret=False):
    body = functools.partial(_reduce_body, fout)
    return pl.pallas_call(
        body,
        grid=(NST,),
        in_specs=[
            pl.BlockSpec((NTL, fout), lambda nt: (nt, 0)),
            pl.BlockSpec((KNB, NTL, FG), lambda nt: (0, nt, 0)),
            pl.BlockSpec((NTL, FG), lambda nt: (nt, 0)),
            pl.BlockSpec((FG, fout), lambda nt: (0, 0)),
        ],
        out_specs=[
            pl.BlockSpec((NTL, fout), lambda nt: (nt, 0)),
            pl.BlockSpec((1, 2, fout), lambda nt: (nt, 0, 0)),
        ],
        out_shape=[
            jax.ShapeDtypeStruct((NPAD, fout), jnp.float32),
            jax.ShapeDtypeStruct((NST, 2, fout), jnp.float32),
        ],
        interpret=interpret,
    )(u, xj, x, wb)


def _final_body(y1_ref, y2_ref, y3_ref, st1_ref, st2_ref, st3_ref,
                g1_ref, be1_ref, g2_ref, be2_ref, g3_ref, be3_ref,
                wf1_ref, bf1_ref, gf_ref, bef_ref, wf2_ref, bf2_ref,
                out_ref, gm1, gm2, gm3):
    b = pl.program_id(0)

    def norm_of(st_ref, g_ref, be_ref, v):
        st = st_ref[...]
        mean = jnp.sum(st[:, 0], axis=0, keepdims=True) / NEDGE
        msq = jnp.sum(st[:, 1], axis=0, keepdims=True) / NEDGE
        var = msq - mean * mean
        return g_ref[...] * (v - mean) / jnp.sqrt(var + 1e-5) + be_ref[...]

    for y_ref, st_ref, g_ref, be_ref, gm in (
            (y1_ref, st1_ref, g1_ref, be1_ref, gm1),
            (y2_ref, st2_ref, g2_ref, be2_ref, gm2),
            (y3_ref, st3_ref, g3_ref, be3_ref, gm3)):
        mx = jnp.max(y_ref[...], axis=0, keepdims=True)
        gn = norm_of(st_ref, g_ref, be_ref, mx)
        rows = lax.broadcasted_iota(jnp.int32, gm.shape, 0)
        gm[...] = jnp.where(rows == b, jnp.broadcast_to(gn, gm.shape),
                            gm[...])

    @pl.when(b == B_ - 1)
    def _():
        wf1 = wf1_ref[...]
        h = (jnp.dot(gm1[...], wf1[0:64], preferred_element_type=jnp.float32)
             + jnp.dot(gm2[...], wf1[64:192],
                       preferred_element_type=jnp.float32)
             + jnp.dot(gm3[...], wf1[192:448],
                       preferred_element_type=jnp.float32)) + bf1_ref[...]
        h = jnp.where(h >= 0, h, 0.2 * h)
        rows = lax.broadcasted_iota(jnp.int32, h.shape, 0)
        valid = rows < B_
        hm = jnp.where(valid, h, 0.0)
        mean = jnp.sum(hm, axis=0, keepdims=True) / float(B_)
        diff = h - mean
        var = jnp.sum(jnp.where(valid, diff * diff, 0.0), axis=0,
                      keepdims=True) / float(B_)
        hn = gf_ref[...] * diff / jnp.sqrt(var + 1e-5) + bef_ref[...]
        out_ref[...] = jnp.dot(hn, wf2_ref[...],
                               preferred_element_type=jnp.float32) + bf2_ref[...]


def _final(y1, y2, y3, st1, st2, st3, g1, be1, g2, be2, g3, be3,
           wf1, bf1, gf, bef, wf2, bf2, *, interpret=False):
    full = lambda shape: pl.BlockSpec(shape, lambda b: tuple(0 for _ in shape))
    return pl.pallas_call(
        _final_body,
        grid=(B_,),
        in_specs=[
            pl.BlockSpec((N_, 64), lambda b: (b, 0)),
            pl.BlockSpec((N_, 128), lambda b: (b, 0)),
            pl.BlockSpec((N_, 256), lambda b: (b, 0)),
            full((NST, 2, 64)), full((NST, 2, 128)), full((NST, 2, 256)),
            full((1, 64)), full((1, 64)),
            full((1, 128)), full((1, 128)),
            full((1, 256)), full((1, 256)),
            full((448, 512)), full((1, 512)), full((1, 512)), full((1, 512)),
            full((512, 128)), full((1, 128)),
        ],
        out_specs=pl.BlockSpec((8, 128), lambda b: (0, 0)),
        out_shape=jax.ShapeDtypeStruct((8, 128), jnp.float32),
        scratch_shapes=[
            pltpu.VMEM((8, 64), jnp.float32),
            pltpu.VMEM((8, 128), jnp.float32),
            pltpu.VMEM((8, 256), jnp.float32),
        ],
        interpret=interpret,
    )(y1, y2, y3, st1, st2, st3, g1, be1, g2, be2, g3, be3,
      wf1, bf1, gf, bef, wf2, bf2)


def _row2(v):
    return v.reshape(1, -1)


def kernel(points, W1, b1, g1, be1, W2, b2, g2, be2, W3, b3, g3, be3,
           Wf1, bf1, gf, bef, Wf2, bf2):
    pos = points[..., :3]
    pos_pad = jnp.zeros((B_, NCOL, 8), jnp.float32).at[:, :N_, :3].set(pos)
    posT = jnp.transpose(pos_pad, (0, 2, 1))
    sq = jnp.sum(pos * pos, axis=-1)
    sq_pad = jnp.zeros((B_, NCOL), jnp.float32).at[:, :N_].set(sq)
    sqr = sq_pad[:, :, None]
    sqc = sq_pad[:, None, :]

    idx = _knn(pos_pad, posT, sqr, sqc)
    idx = idx[:, :N_].reshape(NT, KNB)
    idx_pad = jnp.pad(idx, ((0, NPAD - NT), (0, 0)))
    idx_km = jnp.transpose(idx_pad, (1, 0)).reshape(-1)

    x0p = jnp.pad(points.reshape(NT, 5), ((0, NPAD - NT), (0, FG - 5)))
    dummy_st = jnp.zeros((NST, 2, FG), jnp.float32)
    one = jnp.ones((1, FG), jnp.float32)
    zero = jnp.zeros((1, FG), jnp.float32)

    Wt1p = jnp.pad(W1[:5], ((0, FG - 5), (0, 0)))
    Wb1p = jnp.pad(W1[5:], ((0, FG - 5), (0, 0)))
    Wt2, Wb2 = W2[:64], jnp.pad(W2[64:], ((0, 64), (0, 0)))
    Wt3, Wb3 = W3[:128], W3[128:]

    gather = _make_gather_kernel()

    _, U1 = _normu(x0p, dummy_st, one, zero, Wt1p, _row2(b1),
                   FG, FG, 64, False)
    J1 = gather(idx_km, x0p)
    Y1, ST1 = _reduce(U1, J1, x0p, Wb1p, 64)

    X2, U2 = _normu(Y1, ST1, _row2(g1), _row2(be1), Wt2, _row2(b2),
                    64, FG, 128, True)
    J2 = gather(idx_km, X2)
    Y2, ST2 = _reduce(U2, J2, X2, Wb2, 128)

    X3, U3 = _normu(Y2, ST2, _row2(g2), _row2(be2), Wt3, _row2(b3),
                    128, FG, 256, True)
    J3 = gather(idx_km, X3)
    Y3, ST3 = _reduce(U3, J3, X3, Wb3, 256)

    out8 = _final(Y1[:NT], Y2[:NT], Y3[:NT], ST1, ST2, ST3,
                  _row2(g1), _row2(be1), _row2(g2), _row2(be2),
                  _row2(g3), _row2(be3),
                  Wf1, _row2(bf1), _row2(gf), _row2(bef), Wf2, _row2(bf2))
    return out8[:B_]
